# baseline (speedup 1.0000x reference)
import functools

import jax
import jax.numpy as jnp
from jax.experimental import pallas as pl
from jax.experimental.pallas import tpu as pltpu

B = 4
C_IN = 256
C_SH = 64
N = 5000
HEAD_CLS = (2, 1, 3, 2, 1, 3)
C_OUT = sum(HEAD_CLS)
C_MID = C_SH * len(HEAD_CLS)
EPS = 1e-5
NT = 2560


def _head_body(x_ref, a_ref, ba_ref, w_ref, bw_ref, c_ref, bc_ref, o_ref):
    x = x_ref[0].astype(jnp.bfloat16)
    y = jnp.dot(a_ref[...], x, preferred_element_type=jnp.float32)
    y = jnp.maximum(y + ba_ref[...], 0.0).astype(jnp.bfloat16)
    h = jnp.dot(w_ref[...], y, preferred_element_type=jnp.float32)
    h = jnp.maximum(h + bw_ref[...], 0.0).astype(jnp.bfloat16)
    o = jnp.dot(c_ref[...], h, preferred_element_type=jnp.float32)
    o_ref[0] = o + bc_ref[...]


@functools.partial(jax.jit, static_argnames=())
def kernel(ct_feat, sh_W, sh_b, sh_g, sh_be,
           reg_W1, reg_b1, reg_g1, reg_be1, reg_W2, reg_b2,
           height_W1, height_b1, height_g1, height_be1, height_W2, height_b2,
           dim_W1, dim_b1, dim_g1, dim_be1, dim_W2, dim_b2,
           rot_W1, rot_b1, rot_g1, rot_be1, rot_W2, rot_b2,
           iou_W1, iou_b1, iou_g1, iou_be1, iou_W2, iou_b2,
           hm_W1, hm_b1, hm_g1, hm_be1, hm_W2, hm_b2):
    inv_s = 1.0 / jnp.sqrt(1.0 + EPS)

    a = sh_W * (sh_g * inv_s)[:, None]
    ba = sh_b * sh_g * inv_s + sh_be

    heads = (
        (reg_W1, reg_b1, reg_g1, reg_be1, reg_W2, reg_b2),
        (height_W1, height_b1, height_g1, height_be1, height_W2, height_b2),
        (dim_W1, dim_b1, dim_g1, dim_be1, dim_W2, dim_b2),
        (rot_W1, rot_b1, rot_g1, rot_be1, rot_W2, rot_b2),
        (iou_W1, iou_b1, iou_g1, iou_be1, iou_W2, iou_b2),
        (hm_W1, hm_b1, hm_g1, hm_be1, hm_W2, hm_b2),
    )

    w_all = jnp.concatenate(
        [w1 * (g1 * inv_s)[:, None] for (w1, b1, g1, be1, _, _) in heads], axis=0)
    bw_all = jnp.concatenate(
        [b1 * g1 * inv_s + be1 for (_, b1, g1, be1, _, _) in heads], axis=0)

    c_rows = []
    bc_rows = []
    for i, (_, _, _, _, w2, b2) in enumerate(heads):
        left = jnp.zeros((w2.shape[0], C_SH * i), jnp.float32)
        right = jnp.zeros((w2.shape[0], C_MID - C_SH * (i + 1)), jnp.float32)
        c_rows.append(jnp.concatenate([left, w2, right], axis=1))
        bc_rows.append(b2)
    c = jnp.concatenate(c_rows, axis=0)
    bc = jnp.concatenate(bc_rows, axis=0)

    n_tiles = pl.cdiv(N, NT)
    out = pl.pallas_call(
        _head_body,
        grid=(B, n_tiles),
        in_specs=[
            pl.BlockSpec((1, C_IN, NT), lambda b, j: (b, 0, j)),
            pl.BlockSpec((C_SH, C_IN), lambda b, j: (0, 0)),
            pl.BlockSpec((C_SH, 1), lambda b, j: (0, 0)),
            pl.BlockSpec((C_MID, C_SH), lambda b, j: (0, 0)),
            pl.BlockSpec((C_MID, 1), lambda b, j: (0, 0)),
            pl.BlockSpec((C_OUT, C_MID), lambda b, j: (0, 0)),
            pl.BlockSpec((C_OUT, 1), lambda b, j: (0, 0)),
        ],
        out_specs=pl.BlockSpec((1, C_OUT, NT), lambda b, j: (b, 0, j)),
        out_shape=jax.ShapeDtypeStruct((B, C_OUT, N), jnp.float32),
        compiler_params=pltpu.CompilerParams(
            dimension_semantics=("parallel", "parallel")),
    )(ct_feat, a.astype(jnp.bfloat16), ba[:, None],
      w_all.astype(jnp.bfloat16), bw_all[:, None],
      c.astype(jnp.bfloat16), bc[:, None])
    return out


# speedup vs baseline: 1.0282x; 1.0282x over previous
import functools

import jax
import jax.numpy as jnp
from jax.experimental import pallas as pl
from jax.experimental.pallas import tpu as pltpu

B = 4
C_IN = 256
C_SH = 64
N = 5000
HEAD_CLS = (2, 1, 3, 2, 1, 3)
C_OUT = sum(HEAD_CLS)
C_MID = C_SH * len(HEAD_CLS)
EPS = 1e-5
NT = 5000


def _head_body(x_ref, a_ref, ba_ref, w_ref, bw_ref, c_ref, bc_ref, o_ref):
    x = x_ref[0].astype(jnp.bfloat16)
    y = jnp.dot(a_ref[...], x, preferred_element_type=jnp.float32)
    y = jnp.maximum(y + ba_ref[...], 0.0).astype(jnp.bfloat16)
    h = jnp.dot(w_ref[...], y, preferred_element_type=jnp.float32)
    h = jnp.maximum(h + bw_ref[...], 0.0).astype(jnp.bfloat16)
    o = jnp.dot(c_ref[...], h, preferred_element_type=jnp.float32)
    o_ref[0] = o + bc_ref[...]


@functools.partial(jax.jit, static_argnames=())
def kernel(ct_feat, sh_W, sh_b, sh_g, sh_be,
           reg_W1, reg_b1, reg_g1, reg_be1, reg_W2, reg_b2,
           height_W1, height_b1, height_g1, height_be1, height_W2, height_b2,
           dim_W1, dim_b1, dim_g1, dim_be1, dim_W2, dim_b2,
           rot_W1, rot_b1, rot_g1, rot_be1, rot_W2, rot_b2,
           iou_W1, iou_b1, iou_g1, iou_be1, iou_W2, iou_b2,
           hm_W1, hm_b1, hm_g1, hm_be1, hm_W2, hm_b2):
    inv_s = 1.0 / jnp.sqrt(1.0 + EPS)

    a = sh_W * (sh_g * inv_s)[:, None]
    ba = sh_b * sh_g * inv_s + sh_be

    heads = (
        (reg_W1, reg_b1, reg_g1, reg_be1, reg_W2, reg_b2),
        (height_W1, height_b1, height_g1, height_be1, height_W2, height_b2),
        (dim_W1, dim_b1, dim_g1, dim_be1, dim_W2, dim_b2),
        (rot_W1, rot_b1, rot_g1, rot_be1, rot_W2, rot_b2),
        (iou_W1, iou_b1, iou_g1, iou_be1, iou_W2, iou_b2),
        (hm_W1, hm_b1, hm_g1, hm_be1, hm_W2, hm_b2),
    )

    w_all = jnp.concatenate(
        [w1 * (g1 * inv_s)[:, None] for (w1, b1, g1, be1, _, _) in heads], axis=0)
    bw_all = jnp.concatenate(
        [b1 * g1 * inv_s + be1 for (_, b1, g1, be1, _, _) in heads], axis=0)

    c_rows = []
    bc_rows = []
    for i, (_, _, _, _, w2, b2) in enumerate(heads):
        left = jnp.zeros((w2.shape[0], C_SH * i), jnp.float32)
        right = jnp.zeros((w2.shape[0], C_MID - C_SH * (i + 1)), jnp.float32)
        c_rows.append(jnp.concatenate([left, w2, right], axis=1))
        bc_rows.append(b2)
    c = jnp.concatenate(c_rows, axis=0)
    bc = jnp.concatenate(bc_rows, axis=0)

    out = pl.pallas_call(
        _head_body,
        grid=(B,),
        in_specs=[
            pl.BlockSpec((1, C_IN, N), lambda b: (b, 0, 0)),
            pl.BlockSpec((C_SH, C_IN), lambda b: (0, 0)),
            pl.BlockSpec((C_SH, 1), lambda b: (0, 0)),
            pl.BlockSpec((C_MID, C_SH), lambda b: (0, 0)),
            pl.BlockSpec((C_MID, 1), lambda b: (0, 0)),
            pl.BlockSpec((C_OUT, C_MID), lambda b: (0, 0)),
            pl.BlockSpec((C_OUT, 1), lambda b: (0, 0)),
        ],
        out_specs=pl.BlockSpec((1, C_OUT, N), lambda b: (b, 0, 0)),
        out_shape=jax.ShapeDtypeStruct((B, C_OUT, N), jnp.float32),
        compiler_params=pltpu.CompilerParams(
            dimension_semantics=("parallel",)),
    )(ct_feat, a.astype(jnp.bfloat16), ba[:, None],
      w_all.astype(jnp.bfloat16), bw_all[:, None],
      c.astype(jnp.bfloat16), bc[:, None])
    return out


# speedup vs baseline: 1.1892x; 1.1566x over previous
import functools

import jax
import jax.numpy as jnp
from jax.experimental import pallas as pl
from jax.experimental.pallas import tpu as pltpu

B = 4
C_IN = 256
C_SH = 64
N = 5000
HEAD_CLS = (2, 1, 3, 2, 1, 3)
C_OUT = sum(HEAD_CLS)
C_MID = C_SH * len(HEAD_CLS)
EPS = 1e-5
NT = 5000


def _head_body(x_ref, a_ref, ba_ref, w_ref, bw_ref, c_ref, bc_ref, o_ref):
    x = x_ref[0]
    y = jnp.dot(a_ref[...], x, preferred_element_type=jnp.float32)
    y = jnp.maximum(y + ba_ref[...], 0.0).astype(jnp.bfloat16)
    h = jnp.dot(w_ref[...], y, preferred_element_type=jnp.float32)
    h = jnp.maximum(h + bw_ref[...], 0.0).astype(jnp.bfloat16)
    o = jnp.dot(c_ref[...], h, preferred_element_type=jnp.float32)
    o_ref[0] = o + bc_ref[...]


@functools.partial(jax.jit, static_argnames=())
def kernel(ct_feat, sh_W, sh_b, sh_g, sh_be,
           reg_W1, reg_b1, reg_g1, reg_be1, reg_W2, reg_b2,
           height_W1, height_b1, height_g1, height_be1, height_W2, height_b2,
           dim_W1, dim_b1, dim_g1, dim_be1, dim_W2, dim_b2,
           rot_W1, rot_b1, rot_g1, rot_be1, rot_W2, rot_b2,
           iou_W1, iou_b1, iou_g1, iou_be1, iou_W2, iou_b2,
           hm_W1, hm_b1, hm_g1, hm_be1, hm_W2, hm_b2):
    inv_s = 1.0 / jnp.sqrt(1.0 + EPS)

    a = sh_W * (sh_g * inv_s)[:, None]
    ba = sh_b * sh_g * inv_s + sh_be

    heads = (
        (reg_W1, reg_b1, reg_g1, reg_be1, reg_W2, reg_b2),
        (height_W1, height_b1, height_g1, height_be1, height_W2, height_b2),
        (dim_W1, dim_b1, dim_g1, dim_be1, dim_W2, dim_b2),
        (rot_W1, rot_b1, rot_g1, rot_be1, rot_W2, rot_b2),
        (iou_W1, iou_b1, iou_g1, iou_be1, iou_W2, iou_b2),
        (hm_W1, hm_b1, hm_g1, hm_be1, hm_W2, hm_b2),
    )

    w_all = jnp.concatenate(
        [w1 * (g1 * inv_s)[:, None] for (w1, b1, g1, be1, _, _) in heads], axis=0)
    bw_all = jnp.concatenate(
        [b1 * g1 * inv_s + be1 for (_, b1, g1, be1, _, _) in heads], axis=0)

    c_rows = []
    bc_rows = []
    for i, (_, _, _, _, w2, b2) in enumerate(heads):
        left = jnp.zeros((w2.shape[0], C_SH * i), jnp.float32)
        right = jnp.zeros((w2.shape[0], C_MID - C_SH * (i + 1)), jnp.float32)
        c_rows.append(jnp.concatenate([left, w2, right], axis=1))
        bc_rows.append(b2)
    c = jnp.concatenate(c_rows, axis=0)
    bc = jnp.concatenate(bc_rows, axis=0)

    out = pl.pallas_call(
        _head_body,
        grid=(B,),
        in_specs=[
            pl.BlockSpec((1, C_IN, N), lambda b: (b, 0, 0)),
            pl.BlockSpec((C_SH, C_IN), lambda b: (0, 0)),
            pl.BlockSpec((C_SH, 1), lambda b: (0, 0)),
            pl.BlockSpec((C_MID, C_SH), lambda b: (0, 0)),
            pl.BlockSpec((C_MID, 1), lambda b: (0, 0)),
            pl.BlockSpec((C_OUT, C_MID), lambda b: (0, 0)),
            pl.BlockSpec((C_OUT, 1), lambda b: (0, 0)),
        ],
        out_specs=pl.BlockSpec((1, C_OUT, N), lambda b: (b, 0, 0)),
        out_shape=jax.ShapeDtypeStruct((B, C_OUT, N), jnp.float32),
        compiler_params=pltpu.CompilerParams(
            dimension_semantics=("parallel",)),
    )(ct_feat.astype(jnp.bfloat16), a.astype(jnp.bfloat16), ba[:, None],
      w_all.astype(jnp.bfloat16), bw_all[:, None],
      c.astype(jnp.bfloat16), bc[:, None])
    return out


# speedup vs baseline: 1.2135x; 1.0205x over previous
import functools

import jax
import jax.numpy as jnp
from jax.experimental import pallas as pl
from jax.experimental.pallas import tpu as pltpu

B = 4
C_IN = 256
C_SH = 64
N = 5000
HEAD_CLS = (2, 1, 3, 2, 1, 3)
C_OUT = sum(HEAD_CLS)
C_MID = C_SH * len(HEAD_CLS)
EPS = 1e-5


def _head_body(x_ref, a_ref, ba_ref, w_ref, bw_ref, c_ref, bc_ref, o_ref):
    x = x_ref[0].astype(jnp.bfloat16)
    y = jnp.dot(x, a_ref[...], preferred_element_type=jnp.float32)
    y = jnp.maximum(y + ba_ref[...], 0.0).astype(jnp.bfloat16)
    h = jnp.dot(y, w_ref[...], preferred_element_type=jnp.float32)
    h = jnp.maximum(h + bw_ref[...], 0.0).astype(jnp.bfloat16)
    o = jnp.dot(h, c_ref[...], preferred_element_type=jnp.float32)
    o_ref[0] = o + bc_ref[...]


@functools.partial(jax.jit, static_argnames=())
def kernel(ct_feat, sh_W, sh_b, sh_g, sh_be,
           reg_W1, reg_b1, reg_g1, reg_be1, reg_W2, reg_b2,
           height_W1, height_b1, height_g1, height_be1, height_W2, height_b2,
           dim_W1, dim_b1, dim_g1, dim_be1, dim_W2, dim_b2,
           rot_W1, rot_b1, rot_g1, rot_be1, rot_W2, rot_b2,
           iou_W1, iou_b1, iou_g1, iou_be1, iou_W2, iou_b2,
           hm_W1, hm_b1, hm_g1, hm_be1, hm_W2, hm_b2):
    inv_s = 1.0 / jnp.sqrt(1.0 + EPS)

    at = (sh_W * (sh_g * inv_s)[:, None]).T
    ba = sh_b * sh_g * inv_s + sh_be

    heads = (
        (reg_W1, reg_b1, reg_g1, reg_be1, reg_W2, reg_b2),
        (height_W1, height_b1, height_g1, height_be1, height_W2, height_b2),
        (dim_W1, dim_b1, dim_g1, dim_be1, dim_W2, dim_b2),
        (rot_W1, rot_b1, rot_g1, rot_be1, rot_W2, rot_b2),
        (iou_W1, iou_b1, iou_g1, iou_be1, iou_W2, iou_b2),
        (hm_W1, hm_b1, hm_g1, hm_be1, hm_W2, hm_b2),
    )

    wt = jnp.concatenate(
        [(w1 * (g1 * inv_s)[:, None]).T for (w1, b1, g1, be1, _, _) in heads],
        axis=1)
    bw = jnp.concatenate(
        [b1 * g1 * inv_s + be1 for (_, b1, g1, be1, _, _) in heads], axis=0)

    c_blocks = []
    bc_rows = []
    for i, (_, _, _, _, w2, b2) in enumerate(heads):
        cls = w2.shape[0]
        off = sum(HEAD_CLS[:i])
        c_blocks.append(jnp.pad(w2.T, ((0, 0), (off, C_OUT - off - cls))))
        bc_rows.append(b2)
    ct = jnp.concatenate(c_blocks, axis=0)
    bc = jnp.concatenate(bc_rows, axis=0)

    xt = jnp.swapaxes(ct_feat, 1, 2)
    out = pl.pallas_call(
        _head_body,
        grid=(B,),
        in_specs=[
            pl.BlockSpec((1, N, C_IN), lambda b: (b, 0, 0)),
            pl.BlockSpec((C_IN, C_SH), lambda b: (0, 0)),
            pl.BlockSpec((1, C_SH), lambda b: (0, 0)),
            pl.BlockSpec((C_SH, C_MID), lambda b: (0, 0)),
            pl.BlockSpec((1, C_MID), lambda b: (0, 0)),
            pl.BlockSpec((C_MID, C_OUT), lambda b: (0, 0)),
            pl.BlockSpec((1, C_OUT), lambda b: (0, 0)),
        ],
        out_specs=pl.BlockSpec((1, N, C_OUT), lambda b: (b, 0, 0)),
        out_shape=jax.ShapeDtypeStruct((B, N, C_OUT), jnp.float32),
        compiler_params=pltpu.CompilerParams(
            dimension_semantics=("parallel",)),
    )(xt, at.astype(jnp.bfloat16), ba[None, :],
      wt.astype(jnp.bfloat16), bw[None, :],
      ct.astype(jnp.bfloat16), bc[None, :])
    return jnp.swapaxes(out, 1, 2)
